# Initial kernel scaffold; baseline (speedup 1.0000x reference)
import functools

import jax
import jax.numpy as jnp
from jax.experimental import pallas as pl
from jax.experimental.pallas import tpu as pltpu

_N_HASHES = 4
_QC = 128
_KC = 128


def _attn_body(q_ref, k_ref, v_ref, o_ref, lse_ref):
    q = q_ref[...]
    k = k_ref[...]
    v = v_ref[...]
    inner = jax.lax.dot_general(
        q, k, (((2,), (2,)), ((0,), (0,))), preferred_element_type=jnp.float32
    )
    m = jnp.max(inner, axis=-1, keepdims=True)
    e = jnp.exp(inner - m)
    s = jnp.sum(e, axis=-1, keepdims=True)
    bo = jax.lax.dot_general(
        e, v, (((2,), (1,)), ((0,), (0,))), preferred_element_type=jnp.float32
    )
    o_ref[...] = bo / s
    lse_ref[...] = (m + jnp.log(s))[..., 0]


def _cluster_attention(s_q, s_k, s_v):
    n_cl = s_q.shape[0]
    C = 8
    grid = (n_cl // C,)
    bo, lse = pl.pallas_call(
        _attn_body,
        grid=grid,
        in_specs=[
            pl.BlockSpec((C, _QC, 64), lambda i: (i, 0, 0)),
            pl.BlockSpec((C, _KC, 64), lambda i: (i, 0, 0)),
            pl.BlockSpec((C, _KC, 64), lambda i: (i, 0, 0)),
        ],
        out_specs=[
            pl.BlockSpec((C, _QC, 64), lambda i: (i, 0, 0)),
            pl.BlockSpec((C, _QC), lambda i: (i, 0)),
        ],
        out_shape=[
            jax.ShapeDtypeStruct((n_cl, _QC, 64), jnp.float32),
            jax.ShapeDtypeStruct((n_cl, _QC), jnp.float32),
        ],
    )(s_q, s_k, s_v)
    return bo, lse


def kernel(queries, keys, values, alpha, beta):
    bs, n, d = queries.shape
    v_dim = values.shape[-1]

    q_norms = jnp.linalg.norm(queries, axis=-1, keepdims=True)
    k_norms = jnp.linalg.norm(keys, axis=-1, keepdims=True)
    M = jnp.max(q_norms, axis=1, keepdims=True) + jnp.max(k_norms, axis=1, keepdims=True)
    q_ext = jnp.sqrt(jnp.maximum(M**2 - q_norms**2, 0.0))
    k_ext = jnp.sqrt(jnp.maximum(M**2 - k_norms**2, 0.0))
    a_d = alpha[:d]
    a_q = alpha[d]
    a_k = alpha[d + 1]
    q_hashed = jnp.transpose(queries @ a_d + q_ext * a_q + beta, (2, 0, 1))
    k_hashed = jnp.transpose(keys @ a_d + k_ext * a_k + beta, (2, 0, 1))

    q_positions = jnp.argsort(q_hashed, axis=-1)
    k_positions = jnp.argsort(k_hashed, axis=-1)
    q_rev_positions = jnp.argsort(q_positions, axis=-1)

    offset = jnp.arange(bs)[:, None] * n
    q_flat = (q_positions + offset).reshape(-1)
    k_flat = (k_positions + offset).reshape(-1)

    s_q = jnp.take(queries.reshape(-1, d), q_flat, axis=0).reshape(-1, _QC, d)
    s_k = jnp.take(keys.reshape(-1, d), k_flat, axis=0).reshape(-1, _KC, d)
    s_v = jnp.take(values.reshape(-1, v_dim), k_flat, axis=0).reshape(-1, _KC, v_dim)

    bo, lse = _cluster_attention(s_q, s_k, s_v)
    bo = bo.reshape(_N_HASHES, bs, n, v_dim)
    slogits = lse.reshape(_N_HASHES, bs, n)

    q_offset2 = jnp.arange(bs * _N_HASHES)[:, None] * n
    q_rev_flat = (q_rev_positions.reshape(-1, n) + q_offset2).reshape(-1)
    o = jnp.take(bo.reshape(-1, v_dim), q_rev_flat, axis=0).reshape(
        _N_HASHES, bs, n, v_dim
    )
    logits = jnp.take_along_axis(slogits, q_rev_positions, axis=2)

    probs = jnp.exp(logits - jax.scipy.special.logsumexp(logits, axis=0, keepdims=True))
    return jnp.sum(o * probs[..., None], axis=0)


# speedup vs baseline: 1.0157x; 1.0157x over previous
import functools

import jax
import jax.numpy as jnp
from jax.experimental import pallas as pl
from jax.experimental.pallas import tpu as pltpu

_N_HASHES = 4
_QC = 128
_KC = 128


def _attn_body(q_ref, k_ref, v_ref, o_ref, lse_ref):
    q = q_ref[...]
    k = k_ref[...]
    v = v_ref[...]
    inner = jax.lax.dot_general(
        q, k, (((2,), (2,)), ((0,), (0,))), preferred_element_type=jnp.float32
    )
    m = jnp.max(inner, axis=-1, keepdims=True)
    e = jnp.exp(inner - m)
    s = jnp.sum(e, axis=-1, keepdims=True)
    bo = jax.lax.dot_general(
        e, v, (((2,), (1,)), ((0,), (0,))), preferred_element_type=jnp.float32
    )
    o_ref[...] = bo / s
    lse_ref[...] = (m + jnp.log(s))[..., 0]


def _cluster_attention(s_q, s_k, s_v):
    n_cl = s_q.shape[0]
    C = 8
    grid = (n_cl // C,)
    bo, lse = pl.pallas_call(
        _attn_body,
        grid=grid,
        in_specs=[
            pl.BlockSpec((C, _QC, 64), lambda i: (i, 0, 0)),
            pl.BlockSpec((C, _KC, 64), lambda i: (i, 0, 0)),
            pl.BlockSpec((C, _KC, 64), lambda i: (i, 0, 0)),
        ],
        out_specs=[
            pl.BlockSpec((C, _QC, 64), lambda i: (i, 0, 0)),
            pl.BlockSpec((C, _QC), lambda i: (i, 0)),
        ],
        out_shape=[
            jax.ShapeDtypeStruct((n_cl, _QC, 64), jnp.float32),
            jax.ShapeDtypeStruct((n_cl, _QC), jnp.float32),
        ],
    )(s_q, s_k, s_v)
    return bo, lse


def kernel(queries, keys, values, alpha, beta):
    bs, n, d = queries.shape
    v_dim = values.shape[-1]

    q_norms = jnp.linalg.norm(queries, axis=-1, keepdims=True)
    k_norms = jnp.linalg.norm(keys, axis=-1, keepdims=True)
    M = jnp.max(q_norms, axis=1, keepdims=True) + jnp.max(k_norms, axis=1, keepdims=True)
    q_ext = jnp.sqrt(jnp.maximum(M**2 - q_norms**2, 0.0))
    k_ext = jnp.sqrt(jnp.maximum(M**2 - k_norms**2, 0.0))
    zq = jnp.zeros_like(q_ext)
    Q = jnp.concatenate([queries, q_ext, zq], axis=-1)
    K = jnp.concatenate([keys, zq, k_ext], axis=-1)
    q_hashed = jnp.transpose(Q @ alpha + beta, (2, 0, 1))
    k_hashed = jnp.transpose(K @ alpha + beta, (2, 0, 1))

    q_positions = jnp.argsort(q_hashed, axis=-1)
    k_positions = jnp.argsort(k_hashed, axis=-1)
    q_rev_positions = jnp.argsort(q_positions, axis=-1)

    offset = jnp.arange(bs)[:, None] * n
    q_flat = (q_positions + offset).reshape(-1)
    k_flat = (k_positions + offset).reshape(-1)

    s_q = jnp.take(queries.reshape(-1, d), q_flat, axis=0).reshape(-1, _QC, d)
    s_k = jnp.take(keys.reshape(-1, d), k_flat, axis=0).reshape(-1, _KC, d)
    s_v = jnp.take(values.reshape(-1, v_dim), k_flat, axis=0).reshape(-1, _KC, v_dim)

    bo, lse = _cluster_attention(s_q, s_k, s_v)
    bo = bo.reshape(_N_HASHES, bs, n, v_dim)
    slogits = lse.reshape(_N_HASHES, bs, n)

    q_offset2 = jnp.arange(bs * _N_HASHES)[:, None] * n
    q_rev_flat = (q_rev_positions.reshape(-1, n) + q_offset2).reshape(-1)
    o = jnp.take(bo.reshape(-1, v_dim), q_rev_flat, axis=0).reshape(
        _N_HASHES, bs, n, v_dim
    )
    logits = jnp.take_along_axis(slogits, q_rev_positions, axis=2)

    probs = jnp.exp(logits - jax.scipy.special.logsumexp(logits, axis=0, keepdims=True))
    return jnp.sum(o * probs[..., None], axis=0)


# speedup vs baseline: 11.8649x; 11.6820x over previous
import functools

import jax
import jax.numpy as jnp
from jax import lax
from jax.experimental import pallas as pl
from jax.experimental.pallas import tpu as pltpu
from jax.experimental.pallas import tpu_sc as plsc

_N_HASHES = 4
_QC = 128
_KC = 128

_NC = 2
_NS = 16
_NW = _NC * _NS
_IPD = 128
_DPC = 4
_CHUNK = _IPD * _DPC


def _sc_gather128(table, idx):
    g = idx.shape[0]
    idx2 = idx.reshape(g // _IPD, _IPD)
    rows_per_w = g // _NW
    chunks = rows_per_w // _CHUNK
    mesh = plsc.VectorSubcoreMesh(core_axis_name="c", subcore_axis_name="s")

    @functools.partial(
        pl.kernel,
        mesh=mesh,
        out_type=jax.ShapeDtypeStruct((g, 128), jnp.float32),
        scratch_types=[
            pltpu.VMEM((_DPC, _IPD), jnp.int32),
            pltpu.VMEM((_CHUNK, 128), jnp.float32),
            pltpu.SemaphoreType.DMA,
        ],
    )
    def k(tab_hbm, idx_hbm, out_hbm, idx_v, rows_v, sem):
        wid = lax.axis_index("s") * _NC + lax.axis_index("c")
        row0 = wid * (rows_per_w // _IPD)

        def body(c, _):
            pltpu.sync_copy(idx_hbm.at[pl.ds(row0 + c * _DPC, _DPC)], idx_v)
            cps = []
            for j in range(_DPC):
                cps.append(
                    pltpu.async_copy(
                        tab_hbm.at[idx_v.at[j]],
                        rows_v.at[pl.ds(j * _IPD, _IPD)],
                        sem,
                    )
                )
            for cp in cps:
                cp.wait()
            pltpu.sync_copy(
                rows_v, out_hbm.at[pl.ds(wid * rows_per_w + c * _CHUNK, _CHUNK)]
            )
            return ()

        lax.fori_loop(0, chunks, body, ())

    return k(table, idx2)


def _attn_body(q_ref, kv_ref, o_ref):
    q = q_ref[..., :64]
    k = kv_ref[..., :64]
    v = kv_ref[..., 64:]
    inner = jax.lax.dot_general(
        q, k, (((2,), (2,)), ((0,), (0,))), preferred_element_type=jnp.float32
    )
    m = jnp.max(inner, axis=-1, keepdims=True)
    e = jnp.exp(inner - m)
    s = jnp.sum(e, axis=-1, keepdims=True)
    bo = jax.lax.dot_general(
        e, v, (((2,), (1,)), ((0,), (0,))), preferred_element_type=jnp.float32
    )
    lse = m + jnp.log(s)
    zpad = jnp.zeros_like(bo[..., :63])
    o_ref[...] = jnp.concatenate([bo / s, lse, zpad], axis=-1)


def _cluster_attention(s_q, s_kv):
    n_cl = s_q.shape[0]
    C = 8
    grid = (n_cl // C,)
    obo = pl.pallas_call(
        _attn_body,
        grid=grid,
        in_specs=[
            pl.BlockSpec((C, _QC, 128), lambda i: (i, 0, 0)),
            pl.BlockSpec((C, _KC, 128), lambda i: (i, 0, 0)),
        ],
        out_specs=pl.BlockSpec((C, _QC, 128), lambda i: (i, 0, 0)),
        out_shape=jax.ShapeDtypeStruct((n_cl, _QC, 128), jnp.float32),
    )(s_q, s_kv)
    return obo


def kernel(queries, keys, values, alpha, beta):
    bs, n, d = queries.shape
    v_dim = values.shape[-1]

    q_norms = jnp.linalg.norm(queries, axis=-1, keepdims=True)
    k_norms = jnp.linalg.norm(keys, axis=-1, keepdims=True)
    M = jnp.max(q_norms, axis=1, keepdims=True) + jnp.max(k_norms, axis=1, keepdims=True)
    q_ext = jnp.sqrt(jnp.maximum(M**2 - q_norms**2, 0.0))
    k_ext = jnp.sqrt(jnp.maximum(M**2 - k_norms**2, 0.0))
    zq = jnp.zeros_like(q_ext)
    Q = jnp.concatenate([queries, q_ext, zq], axis=-1)
    K = jnp.concatenate([keys, zq, k_ext], axis=-1)
    q_hashed = jnp.transpose(Q @ alpha + beta, (2, 0, 1))
    k_hashed = jnp.transpose(K @ alpha + beta, (2, 0, 1))

    q_positions = jnp.argsort(q_hashed, axis=-1)
    k_positions = jnp.argsort(k_hashed, axis=-1)
    q_rev_positions = jnp.argsort(q_positions, axis=-1)

    offset = jnp.arange(bs)[:, None] * n
    q_flat = (q_positions + offset).reshape(-1)
    k_flat = (k_positions + offset).reshape(-1)

    q_tab = jnp.concatenate(
        [queries, jnp.zeros((bs, n, 128 - d), jnp.float32)], axis=-1
    ).reshape(-1, 128)
    kv_tab = jnp.concatenate([keys, values], axis=-1).reshape(-1, d + v_dim)

    s_q = _sc_gather128(q_tab, q_flat).reshape(-1, _QC, 128)
    s_kv = _sc_gather128(kv_tab, k_flat).reshape(-1, _KC, 128)

    obo = _cluster_attention(s_q, s_kv)

    q_offset2 = jnp.arange(bs * _N_HASHES)[:, None] * n
    q_rev_flat = (q_rev_positions.reshape(-1, n) + q_offset2).reshape(-1)
    o_l = _sc_gather128(obo.reshape(-1, 128), q_rev_flat).reshape(
        _N_HASHES, bs, n, 128
    )
    o = o_l[..., :v_dim]
    logits = o_l[..., v_dim]

    probs = jnp.exp(logits - jax.scipy.special.logsumexp(logits, axis=0, keepdims=True))
    return jnp.sum(o * probs[..., None], axis=0)


# speedup vs baseline: 11.8817x; 1.0014x over previous
import functools

import jax
import jax.numpy as jnp
from jax import lax
from jax.experimental import pallas as pl
from jax.experimental.pallas import tpu as pltpu
from jax.experimental.pallas import tpu_sc as plsc

_N_HASHES = 4
_QC = 128
_KC = 128

_NC = 2
_NS = 16
_NW = _NC * _NS
_IPD = 128
_DPC = 4
_CHUNK = _IPD * _DPC


def _sc_gather128(table, idx):
    g = idx.shape[0]
    idx2 = idx.reshape(g // _IPD, _IPD)
    rows_per_w = g // _NW
    chunks = rows_per_w // _CHUNK
    mesh = plsc.VectorSubcoreMesh(core_axis_name="c", subcore_axis_name="s")

    @functools.partial(
        pl.kernel,
        mesh=mesh,
        out_type=jax.ShapeDtypeStruct((g, 128), jnp.float32),
        scratch_types=[
            pltpu.VMEM((_DPC, _IPD), jnp.int32),
            pltpu.VMEM((_CHUNK, 128), jnp.float32),
            pltpu.SemaphoreType.DMA,
        ],
    )
    def k(tab_hbm, idx_hbm, out_hbm, idx_v, rows_v, sem):
        wid = lax.axis_index("s") * _NC + lax.axis_index("c")
        row0 = wid * (rows_per_w // _IPD)

        def body(c, _):
            pltpu.sync_copy(idx_hbm.at[pl.ds(row0 + c * _DPC, _DPC)], idx_v)
            cps = []
            for j in range(_DPC):
                cps.append(
                    pltpu.async_copy(
                        tab_hbm.at[idx_v.at[j]],
                        rows_v.at[pl.ds(j * _IPD, _IPD)],
                        sem,
                    )
                )
            for cp in cps:
                cp.wait()
            pltpu.sync_copy(
                rows_v, out_hbm.at[pl.ds(wid * rows_per_w + c * _CHUNK, _CHUNK)]
            )
            return ()

        lax.fori_loop(0, chunks, body, ())

    return k(table, idx2)


def _sc_scatter128(rows, idx):
    g = idx.shape[0]
    idx2 = idx.reshape(g // _IPD, _IPD)
    rows_per_w = g // _NW
    chunks = rows_per_w // _CHUNK
    mesh = plsc.VectorSubcoreMesh(core_axis_name="c", subcore_axis_name="s")

    @functools.partial(
        pl.kernel,
        mesh=mesh,
        out_type=jax.ShapeDtypeStruct((g, 128), jnp.float32),
        scratch_types=[
            pltpu.VMEM((_DPC, _IPD), jnp.int32),
            pltpu.VMEM((_CHUNK, 128), jnp.float32),
            pltpu.SemaphoreType.DMA,
        ],
    )
    def k(rows_hbm, idx_hbm, out_hbm, idx_v, rows_v, sem):
        wid = lax.axis_index("s") * _NC + lax.axis_index("c")
        row0 = wid * (rows_per_w // _IPD)

        def body(c, _):
            pltpu.sync_copy(idx_hbm.at[pl.ds(row0 + c * _DPC, _DPC)], idx_v)
            pltpu.sync_copy(
                rows_hbm.at[pl.ds(wid * rows_per_w + c * _CHUNK, _CHUNK)], rows_v
            )
            cps = []
            for j in range(_DPC):
                cps.append(
                    pltpu.async_copy(
                        rows_v.at[pl.ds(j * _IPD, _IPD)],
                        out_hbm.at[idx_v.at[j]],
                        sem,
                    )
                )
            for cp in cps:
                cp.wait()
            return ()

        lax.fori_loop(0, chunks, body, ())

    return k(rows, idx2)


def _attn_body(q_ref, kv_ref, o_ref):
    q = q_ref[..., :64]
    k = kv_ref[..., :64]
    v = kv_ref[..., 64:]
    inner = jax.lax.dot_general(
        q, k, (((2,), (2,)), ((0,), (0,))), preferred_element_type=jnp.float32
    )
    m = jnp.max(inner, axis=-1, keepdims=True)
    e = jnp.exp(inner - m)
    s = jnp.sum(e, axis=-1, keepdims=True)
    bo = jax.lax.dot_general(
        e, v, (((2,), (1,)), ((0,), (0,))), preferred_element_type=jnp.float32
    )
    lse = m + jnp.log(s)
    zpad = jnp.zeros_like(bo[..., :63])
    o_ref[...] = jnp.concatenate([bo / s, lse, zpad], axis=-1)


def _cluster_attention(s_q, s_kv):
    n_cl = s_q.shape[0]
    C = 8
    grid = (n_cl // C,)
    obo = pl.pallas_call(
        _attn_body,
        grid=grid,
        in_specs=[
            pl.BlockSpec((C, _QC, 128), lambda i: (i, 0, 0)),
            pl.BlockSpec((C, _KC, 128), lambda i: (i, 0, 0)),
        ],
        out_specs=pl.BlockSpec((C, _QC, 128), lambda i: (i, 0, 0)),
        out_shape=jax.ShapeDtypeStruct((n_cl, _QC, 128), jnp.float32),
    )(s_q, s_kv)
    return obo


def kernel(queries, keys, values, alpha, beta):
    bs, n, d = queries.shape
    v_dim = values.shape[-1]

    q_norms = jnp.linalg.norm(queries, axis=-1, keepdims=True)
    k_norms = jnp.linalg.norm(keys, axis=-1, keepdims=True)
    M = jnp.max(q_norms, axis=1, keepdims=True) + jnp.max(k_norms, axis=1, keepdims=True)
    q_ext = jnp.sqrt(jnp.maximum(M**2 - q_norms**2, 0.0))
    k_ext = jnp.sqrt(jnp.maximum(M**2 - k_norms**2, 0.0))
    zq = jnp.zeros_like(q_ext)
    Q = jnp.concatenate([queries, q_ext, zq], axis=-1)
    K = jnp.concatenate([keys, zq, k_ext], axis=-1)
    q_hashed = jnp.transpose(Q @ alpha + beta, (2, 0, 1))
    k_hashed = jnp.transpose(K @ alpha + beta, (2, 0, 1))

    q_positions = jnp.argsort(q_hashed, axis=-1)
    k_positions = jnp.argsort(k_hashed, axis=-1)

    offset = jnp.arange(bs)[:, None] * n
    q_flat = (q_positions + offset).reshape(-1)
    k_flat = (k_positions + offset).reshape(-1)

    q_tab = jnp.concatenate(
        [queries, jnp.zeros((bs, n, 128 - d), jnp.float32)], axis=-1
    ).reshape(-1, 128)
    kv_tab = jnp.concatenate([keys, values], axis=-1).reshape(-1, d + v_dim)

    s_q = _sc_gather128(q_tab, q_flat).reshape(-1, _QC, 128)
    s_kv = _sc_gather128(kv_tab, k_flat).reshape(-1, _KC, 128)

    obo = _cluster_attention(s_q, s_kv)

    q_offset2 = jnp.arange(bs * _N_HASHES)[:, None] * n
    scat_flat = (q_positions.reshape(-1, n) + q_offset2).reshape(-1)
    o_l = _sc_scatter128(obo.reshape(-1, 128), scat_flat).reshape(
        _N_HASHES, bs, n, 128
    )
    o = o_l[..., :v_dim]
    logits = o_l[..., v_dim]

    probs = jnp.exp(logits - jax.scipy.special.logsumexp(logits, axis=0, keepdims=True))
    return jnp.sum(o * probs[..., None], axis=0)


# speedup vs baseline: 13.1922x; 1.1103x over previous
import functools

import jax
import jax.numpy as jnp
from jax import lax
from jax.experimental import pallas as pl
from jax.experimental.pallas import tpu as pltpu
from jax.experimental.pallas import tpu_sc as plsc

_N_HASHES = 4
_QC = 128
_KC = 128

_NC = 2
_NS = 16
_NW = _NC * _NS
_IPD = 128
_DPC = 4
_CHUNK = _IPD * _DPC


def _sc_gather128(table, idx):
    g = idx.shape[0]
    idx2 = idx.reshape(g // _IPD, _IPD)
    rows_per_w = g // _NW
    chunks = rows_per_w // _CHUNK
    mesh = plsc.VectorSubcoreMesh(core_axis_name="c", subcore_axis_name="s")

    @functools.partial(
        pl.kernel,
        mesh=mesh,
        out_type=jax.ShapeDtypeStruct((g, 128), jnp.float32),
        scratch_types=[
            pltpu.VMEM((_DPC, _IPD), jnp.int32),
            pltpu.VMEM((_CHUNK, 128), jnp.float32),
            pltpu.SemaphoreType.DMA,
        ],
    )
    def k(tab_hbm, idx_hbm, out_hbm, idx_v, rows_v, sem):
        wid = lax.axis_index("s") * _NC + lax.axis_index("c")
        row0 = wid * (rows_per_w // _IPD)

        def body(c, _):
            pltpu.sync_copy(idx_hbm.at[pl.ds(row0 + c * _DPC, _DPC)], idx_v)
            cps = []
            for j in range(_DPC):
                cps.append(
                    pltpu.async_copy(
                        tab_hbm.at[idx_v.at[j]],
                        rows_v.at[pl.ds(j * _IPD, _IPD)],
                        sem,
                    )
                )
            for cp in cps:
                cp.wait()
            pltpu.sync_copy(
                rows_v, out_hbm.at[pl.ds(wid * rows_per_w + c * _CHUNK, _CHUNK)]
            )
            return ()

        lax.fori_loop(0, chunks, body, ())

    return k(table, idx2)


def _sc_scatter128(rows, idx):
    g = idx.shape[0]
    idx2 = idx.reshape(g // _IPD, _IPD)
    rows_per_w = g // _NW
    chunks = rows_per_w // _CHUNK
    mesh = plsc.VectorSubcoreMesh(core_axis_name="c", subcore_axis_name="s")

    @functools.partial(
        pl.kernel,
        mesh=mesh,
        out_type=jax.ShapeDtypeStruct((g, 128), jnp.float32),
        scratch_types=[
            pltpu.VMEM((_DPC, _IPD), jnp.int32),
            pltpu.VMEM((_CHUNK, 128), jnp.float32),
            pltpu.SemaphoreType.DMA,
        ],
    )
    def k(rows_hbm, idx_hbm, out_hbm, idx_v, rows_v, sem):
        wid = lax.axis_index("s") * _NC + lax.axis_index("c")
        row0 = wid * (rows_per_w // _IPD)

        def body(c, _):
            pltpu.sync_copy(idx_hbm.at[pl.ds(row0 + c * _DPC, _DPC)], idx_v)
            pltpu.sync_copy(
                rows_hbm.at[pl.ds(wid * rows_per_w + c * _CHUNK, _CHUNK)], rows_v
            )
            cps = []
            for j in range(_DPC):
                cps.append(
                    pltpu.async_copy(
                        rows_v.at[pl.ds(j * _IPD, _IPD)],
                        out_hbm.at[idx_v.at[j]],
                        sem,
                    )
                )
            for cp in cps:
                cp.wait()
            return ()

        lax.fori_loop(0, chunks, body, ())

    return k(rows, idx2)


def _attn_body(q_ref, kv_ref, o_ref):
    q = q_ref[..., :64]
    k = kv_ref[..., :64]
    v = kv_ref[..., 64:]
    inner = jax.lax.dot_general(
        q, k, (((2,), (2,)), ((0,), (0,))), preferred_element_type=jnp.float32
    )
    m = jnp.max(inner, axis=-1, keepdims=True)
    e = jnp.exp(inner - m)
    s = jnp.sum(e, axis=-1, keepdims=True)
    bo = jax.lax.dot_general(
        e, v, (((2,), (1,)), ((0,), (0,))), preferred_element_type=jnp.float32
    )
    lse = m + jnp.log(s)
    zpad = jnp.zeros_like(bo[..., :63])
    o_ref[...] = jnp.concatenate([bo / s, lse, zpad], axis=-1)


def _cluster_attention(s_q, s_kv):
    n_cl = s_q.shape[0]
    C = 8
    grid = (n_cl // C,)
    obo = pl.pallas_call(
        _attn_body,
        grid=grid,
        in_specs=[
            pl.BlockSpec((C, _QC, 128), lambda i: (i, 0, 0)),
            pl.BlockSpec((C, _KC, 128), lambda i: (i, 0, 0)),
        ],
        out_specs=pl.BlockSpec((C, _QC, 128), lambda i: (i, 0, 0)),
        out_shape=jax.ShapeDtypeStruct((n_cl, _QC, 128), jnp.float32),
    )(s_q, s_kv)
    return obo


def kernel(queries, keys, values, alpha, beta):
    bs, n, d = queries.shape
    v_dim = values.shape[-1]

    q_norms = jnp.linalg.norm(queries, axis=-1, keepdims=True)
    k_norms = jnp.linalg.norm(keys, axis=-1, keepdims=True)
    M = jnp.max(q_norms, axis=1, keepdims=True) + jnp.max(k_norms, axis=1, keepdims=True)
    q_ext = jnp.sqrt(jnp.maximum(M**2 - q_norms**2, 0.0))
    k_ext = jnp.sqrt(jnp.maximum(M**2 - k_norms**2, 0.0))
    zq = jnp.zeros_like(q_ext)
    Q = jnp.concatenate([queries, q_ext, zq], axis=-1)
    K = jnp.concatenate([keys, zq, k_ext], axis=-1)
    q_hashed = jnp.transpose(Q @ alpha + beta, (2, 0, 1))
    k_hashed = jnp.transpose(K @ alpha + beta, (2, 0, 1))

    q_positions = jnp.argsort(q_hashed, axis=-1)
    k_positions = jnp.argsort(k_hashed, axis=-1)

    offset = jnp.arange(bs)[:, None] * n

    q_tab = jnp.concatenate(
        [queries, jnp.zeros((bs, n, 128 - d), jnp.float32)], axis=-1
    ).reshape(-1, 128)
    kv_tab = jnp.concatenate([keys, values], axis=-1).reshape(-1, d + v_dim)

    o_l_rounds = []
    for h in range(_N_HASHES):
        q_flat = (q_positions[h] + offset).reshape(-1)
        k_flat = (k_positions[h] + offset).reshape(-1)
        s_q = _sc_gather128(q_tab, q_flat).reshape(-1, _QC, 128)
        s_kv = _sc_gather128(kv_tab, k_flat).reshape(-1, _KC, 128)
        obo = _cluster_attention(s_q, s_kv)
        o_l_rounds.append(_sc_scatter128(obo.reshape(-1, 128), q_flat))
    o_l = jnp.stack(o_l_rounds).reshape(_N_HASHES, bs, n, 128)
    o = o_l[..., :v_dim]
    logits = o_l[..., v_dim]

    probs = jnp.exp(logits - jax.scipy.special.logsumexp(logits, axis=0, keepdims=True))
    return jnp.sum(o * probs[..., None], axis=0)


# speedup vs baseline: 13.8578x; 1.0505x over previous
import functools

import jax
import jax.numpy as jnp
from jax import lax
from jax.experimental import pallas as pl
from jax.experimental.pallas import tpu as pltpu
from jax.experimental.pallas import tpu_sc as plsc

_N_HASHES = 4
_QC = 128
_KC = 128

_NC = 2
_NS = 16
_NW = _NC * _NS
_IPD = 128
_DPC = 2
_CHUNK = _IPD * _DPC


def _sc_gather128(table, idx):
    g = idx.shape[0]
    idx2 = idx.reshape(g // _IPD, _IPD)
    rows_per_w = g // _NW
    chunks = rows_per_w // _CHUNK
    mesh = plsc.VectorSubcoreMesh(core_axis_name="c", subcore_axis_name="s")

    @functools.partial(
        pl.kernel,
        mesh=mesh,
        out_type=jax.ShapeDtypeStruct((g, 128), jnp.float32),
        scratch_types=[
            pltpu.VMEM((_DPC, _IPD), jnp.int32),
            pltpu.VMEM((_DPC, _IPD), jnp.int32),
            pltpu.VMEM((_CHUNK, 128), jnp.float32),
            pltpu.VMEM((_CHUNK, 128), jnp.float32),
            pltpu.SemaphoreType.DMA,
            pltpu.SemaphoreType.DMA,
            pltpu.SemaphoreType.DMA,
            pltpu.SemaphoreType.DMA,
        ],
    )
    def k(tab_hbm, idx_hbm, out_hbm, i0, i1, r0, r1, sg0, sg1, sw0, sw1):
        wid = lax.axis_index("s") * _NC + lax.axis_index("c")
        row0 = wid * (rows_per_w // _IPD)
        base = wid * rows_per_w
        idxv = (i0, i1)
        rows = (r0, r1)
        semg = (sg0, sg1)
        semw = (sw0, sw1)
        gcps = [None, None]
        wcps = [None, None]
        for c in range(chunks):
            b = c % 2
            if wcps[b] is not None:
                wcps[b].wait()
            pltpu.sync_copy(idx_hbm.at[pl.ds(row0 + c * _DPC, _DPC)], idxv[b])
            gcps[b] = [
                pltpu.async_copy(
                    tab_hbm.at[idxv[b].at[j]],
                    rows[b].at[pl.ds(j * _IPD, _IPD)],
                    semg[b],
                )
                for j in range(_DPC)
            ]
            if c >= 1:
                p = 1 - b
                for cp in gcps[p]:
                    cp.wait()
                wcps[p] = pltpu.async_copy(
                    rows[p],
                    out_hbm.at[pl.ds(base + (c - 1) * _CHUNK, _CHUNK)],
                    semw[p],
                )
        last = (chunks - 1) % 2
        for cp in gcps[last]:
            cp.wait()
        wcps[last] = pltpu.async_copy(
            rows[last],
            out_hbm.at[pl.ds(base + (chunks - 1) * _CHUNK, _CHUNK)],
            semw[last],
        )
        wcps[1 - last].wait()
        wcps[last].wait()

    return k(table, idx2)


def _sc_scatter128(rows_in, idx):
    g = idx.shape[0]
    idx2 = idx.reshape(g // _IPD, _IPD)
    rows_per_w = g // _NW
    chunks = rows_per_w // _CHUNK
    mesh = plsc.VectorSubcoreMesh(core_axis_name="c", subcore_axis_name="s")

    @functools.partial(
        pl.kernel,
        mesh=mesh,
        out_type=jax.ShapeDtypeStruct((g, 128), jnp.float32),
        scratch_types=[
            pltpu.VMEM((_DPC, _IPD), jnp.int32),
            pltpu.VMEM((_DPC, _IPD), jnp.int32),
            pltpu.VMEM((_CHUNK, 128), jnp.float32),
            pltpu.VMEM((_CHUNK, 128), jnp.float32),
            pltpu.SemaphoreType.DMA,
            pltpu.SemaphoreType.DMA,
            pltpu.SemaphoreType.DMA,
            pltpu.SemaphoreType.DMA,
        ],
    )
    def k(rows_hbm, idx_hbm, out_hbm, i0, i1, r0, r1, sr0, sr1, ss0, ss1):
        wid = lax.axis_index("s") * _NC + lax.axis_index("c")
        row0 = wid * (rows_per_w // _IPD)
        base = wid * rows_per_w
        idxv = (i0, i1)
        rows = (r0, r1)
        semr = (sr0, sr1)
        sems = (ss0, ss1)
        rcps = [None, None]
        scps = [None, None]
        for c in range(chunks):
            b = c % 2
            if scps[b] is not None:
                for cp in scps[b]:
                    cp.wait()
            pltpu.sync_copy(idx_hbm.at[pl.ds(row0 + c * _DPC, _DPC)], idxv[b])
            rcps[b] = pltpu.async_copy(
                rows_hbm.at[pl.ds(base + c * _CHUNK, _CHUNK)], rows[b], semr[b]
            )
            if c >= 1:
                p = 1 - b
                rcps[p].wait()
                scps[p] = [
                    pltpu.async_copy(
                        rows[p].at[pl.ds(j * _IPD, _IPD)],
                        out_hbm.at[idxv[p].at[j]],
                        sems[p],
                    )
                    for j in range(_DPC)
                ]
        last = (chunks - 1) % 2
        rcps[last].wait()
        scps[last] = [
            pltpu.async_copy(
                rows[last].at[pl.ds(j * _IPD, _IPD)],
                out_hbm.at[idxv[last].at[j]],
                sems[last],
            )
            for j in range(_DPC)
        ]
        for b in (1 - last, last):
            for cp in scps[b]:
                cp.wait()

    return k(rows_in, idx2)


def _attn_body(q_ref, kv_ref, o_ref):
    q = q_ref[..., :64]
    k = kv_ref[..., :64]
    v = kv_ref[..., 64:]
    inner = jax.lax.dot_general(
        q, k, (((2,), (2,)), ((0,), (0,))), preferred_element_type=jnp.float32
    )
    m = jnp.max(inner, axis=-1, keepdims=True)
    e = jnp.exp(inner - m)
    s = jnp.sum(e, axis=-1, keepdims=True)
    bo = jax.lax.dot_general(
        e, v, (((2,), (1,)), ((0,), (0,))), preferred_element_type=jnp.float32
    )
    lse = m + jnp.log(s)
    zpad = jnp.zeros_like(bo[..., :63])
    o_ref[...] = jnp.concatenate([bo / s, lse, zpad], axis=-1)


def _cluster_attention(s_q, s_kv):
    n_cl = s_q.shape[0]
    C = 16
    grid = (n_cl // C,)
    obo = pl.pallas_call(
        _attn_body,
        grid=grid,
        in_specs=[
            pl.BlockSpec((C, _QC, 128), lambda i: (i, 0, 0)),
            pl.BlockSpec((C, _KC, 128), lambda i: (i, 0, 0)),
        ],
        out_specs=pl.BlockSpec((C, _QC, 128), lambda i: (i, 0, 0)),
        out_shape=jax.ShapeDtypeStruct((n_cl, _QC, 128), jnp.float32),
    )(s_q, s_kv)
    return obo


def kernel(queries, keys, values, alpha, beta):
    bs, n, d = queries.shape
    v_dim = values.shape[-1]

    q_norms = jnp.linalg.norm(queries, axis=-1, keepdims=True)
    k_norms = jnp.linalg.norm(keys, axis=-1, keepdims=True)
    M = jnp.max(q_norms, axis=1, keepdims=True) + jnp.max(k_norms, axis=1, keepdims=True)
    q_ext = jnp.sqrt(jnp.maximum(M**2 - q_norms**2, 0.0))
    k_ext = jnp.sqrt(jnp.maximum(M**2 - k_norms**2, 0.0))
    zq = jnp.zeros_like(q_ext)
    Q = jnp.concatenate([queries, q_ext, zq], axis=-1)
    K = jnp.concatenate([keys, zq, k_ext], axis=-1)
    q_hashed = jnp.transpose(Q @ alpha + beta, (2, 0, 1))
    k_hashed = jnp.transpose(K @ alpha + beta, (2, 0, 1))

    offset = jnp.arange(bs)[:, None] * n

    q_tab = jnp.concatenate(
        [queries, jnp.zeros((bs, n, 128 - d), jnp.float32)], axis=-1
    ).reshape(-1, 128)
    kv_tab = jnp.concatenate([keys, values], axis=-1).reshape(-1, d + v_dim)

    o_l_rounds = []
    for h in range(_N_HASHES):
        q_positions = jnp.argsort(q_hashed[h], axis=-1)
        k_positions = jnp.argsort(k_hashed[h], axis=-1)
        q_flat = (q_positions + offset).reshape(-1)
        k_flat = (k_positions + offset).reshape(-1)
        s_q = _sc_gather128(q_tab, q_flat).reshape(-1, _QC, 128)
        s_kv = _sc_gather128(kv_tab, k_flat).reshape(-1, _KC, 128)
        obo = _cluster_attention(s_q, s_kv)
        o_l_rounds.append(_sc_scatter128(obo.reshape(-1, 128), q_flat))
    o_l = jnp.stack(o_l_rounds).reshape(_N_HASHES, bs, n, 128)
    o = o_l[..., :v_dim]
    logits = o_l[..., v_dim]

    probs = jnp.exp(logits - jax.scipy.special.logsumexp(logits, axis=0, keepdims=True))
    return jnp.sum(o * probs[..., None], axis=0)
